# baseline (speedup 1.0000x reference)
import functools

import jax
import jax.numpy as jnp
from jax import lax
from jax.experimental import pallas as pl
from jax.experimental.pallas import tpu as pltpu
from jax.experimental.pallas import tpu_sc as plsc

D_MODEL = 64
NUM_WORKERS = 32
CHUNK = 128
RBLK = 128

_mesh = plsc.VectorSubcoreMesh(core_axis_name="c", subcore_axis_name="s")


def _build_repack(vocab: int):
    n_blocks = vocab // RBLK
    tail = vocab - n_blocks * RBLK

    @functools.partial(
        pl.kernel,
        out_type=jax.ShapeDtypeStruct((vocab, 2 * D_MODEL), jnp.float32),
        mesh=_mesh,
        scratch_types=[
            [pltpu.VMEM((D_MODEL, RBLK), jnp.float32) for _ in range(2)],
            [pltpu.VMEM((RBLK, 2 * D_MODEL), jnp.float32) for _ in range(2)],
            pltpu.VMEM((D_MODEL, 64), jnp.float32),
            pltpu.VMEM((64, 2 * D_MODEL), jnp.float32),
            [pltpu.SemaphoreType.DMA for _ in range(2)],
            [pltpu.SemaphoreType.DMA for _ in range(2)],
        ],
        compiler_params=pltpu.CompilerParams(needs_layout_passes=False),
    )
    def repack_kernel(tab_t_hbm, big_hbm, slab, tslab, tslab_in, tslab_out,
                      rsem, wsem):
        wid = lax.axis_index("s") * 2 + lax.axis_index("c")
        nt = jnp.where(wid < n_blocks % NUM_WORKERS,
                       n_blocks // NUM_WORKERS + 1,
                       n_blocks // NUM_WORKERS).astype(jnp.int32)

        def fire_read(t, b):
            blk = wid + t * NUM_WORKERS
            pltpu.async_copy(
                tab_t_hbm.at[:, pl.ds(blk * RBLK, RBLK)], slab[b], rsem[b])

        def drain_read(t, b):
            blk = wid + t * NUM_WORKERS
            pltpu.make_async_copy(
                tab_t_hbm.at[:, pl.ds(blk * RBLK, RBLK)], slab[b],
                rsem[b]).wait()

        def fire_write(t, b):
            blk = wid + t * NUM_WORKERS
            pltpu.async_copy(
                tslab[b], big_hbm.at[pl.ds(blk * RBLK, RBLK)], wsem[b])

        def drain_write(t, b):
            blk = wid + t * NUM_WORKERS
            pltpu.make_async_copy(
                tslab[b], big_hbm.at[pl.ds(blk * RBLK, RBLK)], wsem[b]).wait()

        iota16 = lax.iota(jnp.int32, 16)

        fire_read(0, 0)

        @pl.when(nt > 1)
        def _():
            fire_read(1, 1)

        @pl.loop(0, 246, step=2)
        def _(t_base):
            for b in range(2):
                t = t_base + b

                @pl.when(t < nt)
                def _():
                    drain_read(t, b)

                    @pl.when(t >= 2)
                    def _():
                        drain_write(t - 2, b)

                    for c in range(RBLK):
                        cv = jnp.zeros((16,), jnp.int32) + c
                        for j0 in range(0, D_MODEL, 16):
                            tslab[b][c, pl.ds(j0, 16)] = plsc.load_gather(
                                slab[b], [iota16 + j0, cv])
                    fire_write(t, b)

                    @pl.when(t + 2 < nt)
                    def _():
                        fire_read(t + 2, b)

        for b in range(2):
            tl = nt - 1 - lax.rem(nt - 1 + b + 1, 2)

            @pl.when(tl >= 0)
            def _():
                drain_write(tl, b)

        if tail:
            @pl.when(wid == 0)
            def _():
                pltpu.sync_copy(
                    tab_t_hbm.at[:, pl.ds(n_blocks * RBLK, tail)], tslab_in)
                for c in range(tail):
                    cv = jnp.zeros((16,), jnp.int32) + c
                    for j0 in range(0, D_MODEL, 16):
                        tslab_out[c, pl.ds(j0, 16)] = plsc.load_gather(
                            tslab_in, [iota16 + j0, cv])
                pltpu.sync_copy(
                    tslab_out, big_hbm.at[pl.ds(n_blocks * RBLK, tail)])

    return repack_kernel


def _build_gather(batch: int, vocab: int):
    assert batch % NUM_WORKERS == 0
    b_per_w = batch // NUM_WORKERS
    assert b_per_w % CHUNK == 0
    n_chunks = b_per_w // CHUNK

    @functools.partial(
        pl.kernel,
        out_type=jax.ShapeDtypeStruct((batch, D_MODEL), jnp.float32),
        mesh=_mesh,
        scratch_types=[
            pltpu.VMEM((b_per_w,), jnp.int32),
            [pltpu.VMEM((CHUNK, 2 * D_MODEL), jnp.float32) for _ in range(2)],
            [pltpu.VMEM((CHUNK, D_MODEL), jnp.float32) for _ in range(2)],
            [pltpu.SemaphoreType.DMA for _ in range(2)],
            [pltpu.SemaphoreType.DMA for _ in range(2)],
        ],
    )
    def gather_kernel(big_hbm, idx_hbm, out_hbm, idx_v, wide, outb, gsem,
                      wsem):
        wid = lax.axis_index("s") * 2 + lax.axis_index("c")
        base = wid * b_per_w
        pltpu.sync_copy(idx_hbm.at[pl.ds(base, b_per_w)], idx_v)

        def fire_gather(t, b):
            pltpu.async_copy(
                big_hbm.at[idx_v.at[pl.ds(t * CHUNK, CHUNK)]], wide[b],
                gsem[b])

        def drain_gather(t, b):
            pltpu.make_async_copy(
                big_hbm.at[idx_v.at[pl.ds(t * CHUNK, CHUNK)]], wide[b],
                gsem[b]).wait()

        def fire_write(t, b):
            pltpu.async_copy(
                outb[b], out_hbm.at[pl.ds(base + t * CHUNK, CHUNK)], wsem[b])

        def drain_write(t, b):
            pltpu.make_async_copy(
                outb[b], out_hbm.at[pl.ds(base + t * CHUNK, CHUNK)],
                wsem[b]).wait()

        fire_gather(0, 0)

        @pl.loop(0, n_chunks, step=2)
        def _(t_base):
            for b in range(2):
                t = t_base + b
                drain_gather(t, b)

                @pl.when(t + 1 < n_chunks)
                def _():
                    fire_gather(t + 1, 1 - b)

                @pl.when(t >= 2)
                def _():
                    drain_write(t - 2, b)

                for c in range(CHUNK):
                    for j0 in range(0, D_MODEL, 16):
                        outb[b][c, pl.ds(j0, 16)] = wide[b][c, pl.ds(j0, 16)]
                fire_write(t, b)

        drain_write(n_chunks - 2, 0)
        drain_write(n_chunks - 1, 1)

    return gather_kernel


def kernel(x, emb_table):
    b, s = x.shape
    vocab = emb_table.shape[0]
    flat_idx = x.reshape(b * s).astype(jnp.int32)
    big = _build_repack(vocab)(emb_table.T)
    out = _build_gather(b * s, vocab)(big, flat_idx)
    return out.reshape(b, s, D_MODEL)


# speedup vs baseline: 1.6554x; 1.6554x over previous
import functools

import jax
import jax.numpy as jnp
from jax import lax
from jax.experimental import pallas as pl
from jax.experimental.pallas import tpu as pltpu
from jax.experimental.pallas import tpu_sc as plsc

D_MODEL = 64
NUM_WORKERS = 32
SUPER = 400
NBUF = 4


def _build_kernel(batch: int):
    assert batch % NUM_WORKERS == 0
    b_per_w = batch // NUM_WORKERS
    assert b_per_w % SUPER == 0
    n_super = b_per_w // SUPER
    assert n_super % NBUF == 0

    mesh = plsc.VectorSubcoreMesh(core_axis_name="c", subcore_axis_name="s")

    @functools.partial(
        pl.kernel,
        out_type=jax.ShapeDtypeStruct((batch, D_MODEL), jnp.float32),
        mesh=mesh,
        scratch_types=[
            pltpu.VMEM((b_per_w,), jnp.int32),
            [pltpu.VMEM((SUPER, D_MODEL), jnp.float32) for _ in range(NBUF)],
            [pltpu.SemaphoreType.DMA for _ in range(NBUF)],
            [pltpu.SemaphoreType.DMA for _ in range(NBUF)],
        ],
        compiler_params=pltpu.CompilerParams(use_tc_tiling_on_sc=False),
    )
    def emb_kernel(table_hbm, idx_hbm, out_hbm, idx_v, rows, gsem, wsem):
        wid = lax.axis_index("s") * 2 + lax.axis_index("c")
        base = wid * b_per_w
        pltpu.sync_copy(idx_hbm.at[pl.ds(base, b_per_w)], idx_v)

        def fire_gather(t, b):
            pltpu.async_copy(
                table_hbm.at[idx_v.at[pl.ds(t * SUPER, SUPER)]], rows[b], gsem[b]
            )

        def drain_gather(t, b):
            pltpu.make_async_copy(
                table_hbm.at[idx_v.at[pl.ds(t * SUPER, SUPER)]], rows[b], gsem[b]
            ).wait()

        def fire_write(t, b):
            pltpu.async_copy(
                rows[b], out_hbm.at[pl.ds(base + t * SUPER, SUPER)], wsem[b]
            )

        def drain_write(t, b):
            pltpu.make_async_copy(
                rows[b], out_hbm.at[pl.ds(base + t * SUPER, SUPER)], wsem[b]
            ).wait()

        fire_gather(0, 0)
        fire_gather(1, 1)

        @pl.loop(0, n_super, step=NBUF)
        def _(t_base):
            for b in range(NBUF):
                t = t_base + b
                drain_gather(t, b)
                fire_write(t, b)
                b2 = (b + 2) % NBUF

                @pl.when(t >= 2)
                def _():
                    drain_write(t - 2, b2)

                @pl.when(t + 2 < n_super)
                def _():
                    fire_gather(t + 2, b2)

        drain_write(n_super - 2, (n_super - 2) % NBUF)
        drain_write(n_super - 1, (n_super - 1) % NBUF)

    return emb_kernel


def kernel(x, emb_table):
    b, s = x.shape
    flat_idx = x.reshape(b * s).astype(jnp.int32)
    out = _build_kernel(b * s)(emb_table, flat_idx)
    return out.reshape(b, s, D_MODEL)
